# baseline (speedup 1.0000x reference)
import functools

import jax
import jax.numpy as jnp
from jax import lax
from jax.experimental import pallas as pl
from jax.experimental.pallas import tpu as pltpu
from jax.experimental.pallas import tpu_sc as plsc

NUM_WORKERS = 32
IDX_CHUNK = 128


def _gather_pair(uid2, iid2, user_table, item_table):
    n_rows_total, chunk = uid2.shape
    batch = n_rows_total * chunk
    embed = user_table.shape[1]
    rows_per_w = batch // NUM_WORKERS
    nch = rows_per_w // chunk

    mesh = plsc.VectorSubcoreMesh(core_axis_name="c", subcore_axis_name="s")

    @functools.partial(
        pl.kernel,
        mesh=mesh,
        out_type=(
            jax.ShapeDtypeStruct((batch, embed), jnp.float32),
            jax.ShapeDtypeStruct((batch, embed), jnp.float32),
        ),
        scratch_types=[
            pltpu.VMEM((nch, chunk), jnp.int32),
            pltpu.VMEM((nch, chunk), jnp.int32),
            pltpu.VMEM((rows_per_w, embed), jnp.float32),
            pltpu.SemaphoreType.DMA,
        ],
    )
    def gather_kernel(uid_hbm, iid_hbm, ut_hbm, it_hbm, out_u, out_i,
                      uidx_v, iidx_v, rows_v, sem):
        wid = lax.axis_index("s") * 2 + lax.axis_index("c")
        base = wid * rows_per_w
        idx_row = wid * nch
        pltpu.sync_copy(uid_hbm.at[pl.ds(idx_row, nch)], uidx_v)
        pltpu.sync_copy(iid_hbm.at[pl.ds(idx_row, nch)], iidx_v)
        cps = [
            pltpu.async_copy(ut_hbm.at[uidx_v.at[j]],
                             rows_v.at[pl.ds(j * chunk, chunk)], sem)
            for j in range(nch)
        ]
        for c in cps:
            c.wait()
        pltpu.sync_copy(rows_v, out_u.at[pl.ds(base, rows_per_w)])
        cps = [
            pltpu.async_copy(it_hbm.at[iidx_v.at[j]],
                             rows_v.at[pl.ds(j * chunk, chunk)], sem)
            for j in range(nch)
        ]
        for c in cps:
            c.wait()
        pltpu.sync_copy(rows_v, out_i.at[pl.ds(base, rows_per_w)])

    return gather_kernel(uid2, iid2, user_table, item_table)


def _mlp_body(ue_ref, ie_ref, w1_ref, b1_ref, w2_ref, b2_ref,
              w3_ref, b3_ref, o_ref):
    embed = ue_ref.shape[1]
    h = jnp.dot(ue_ref[...], w1_ref[0:embed, :],
                preferred_element_type=jnp.float32)
    h = h + jnp.dot(ie_ref[...], w1_ref[embed:2 * embed, :],
                    preferred_element_type=jnp.float32)
    h1 = jnp.maximum(h + b1_ref[...], 0.0)
    h2 = jnp.maximum(
        jnp.dot(h1, w2_ref[...], preferred_element_type=jnp.float32)
        + b2_ref[...], 0.0)
    o = jnp.maximum(
        jnp.dot(h2, w3_ref[...], preferred_element_type=jnp.float32)
        + b3_ref[...], 0.0)
    o_ref[...] = o[:, 0]


def _mlp(ue, ie, w1, b1, w2, b2, w3, b3, block=2048):
    batch, embed = ue.shape
    grid = batch // block
    full = lambda shape: pl.BlockSpec(shape, lambda i: (0, 0))
    return pl.pallas_call(
        _mlp_body,
        grid=(grid,),
        in_specs=[
            pl.BlockSpec((block, embed), lambda i: (i, 0)),
            pl.BlockSpec((block, embed), lambda i: (i, 0)),
            full(w1.shape),
            full(b1.shape),
            full(w2.shape),
            full(b2.shape),
            full(w3.shape),
            full(b3.shape),
        ],
        out_specs=pl.BlockSpec((block,), lambda i: (i,)),
        out_shape=jax.ShapeDtypeStruct((batch,), jnp.float32),
    )(ue, ie, w1, b1, w2, b2, w3, b3)


NUM_CHUNKS = 2


def kernel(user_id, item_id, user_table, item_table, W1, b1, W2, b2, W3, b3):
    batch = user_id.shape[0]
    uid2 = user_id.astype(jnp.int32).reshape(batch // IDX_CHUNK, IDX_CHUNK)
    iid2 = item_id.astype(jnp.int32).reshape(batch // IDX_CHUNK, IDX_CHUNK)
    b1r = b1.reshape(1, -1)
    b2r = b2.reshape(1, -1)
    b3r = b3.reshape(1, 1)
    n_rows = uid2.shape[0]
    rows_per_chunk = n_rows // NUM_CHUNKS
    outs = []
    for c in range(NUM_CHUNKS):
        sl = slice(c * rows_per_chunk, (c + 1) * rows_per_chunk)
        ue, ie = _gather_pair(uid2[sl], iid2[sl], user_table, item_table)
        outs.append(_mlp(ue, ie, W1, b1r, W2, b2r, W3, b3r))
    return jnp.concatenate(outs) if NUM_CHUNKS > 1 else outs[0]


# speedup vs baseline: 1.0554x; 1.0554x over previous
import functools

import jax
import jax.numpy as jnp
from jax import lax
from jax.experimental import pallas as pl
from jax.experimental.pallas import tpu as pltpu
from jax.experimental.pallas import tpu_sc as plsc

NUM_WORKERS = 32
IDX_CHUNK = 128


def _gather_pair(uid2, iid2, user_table, item_table):
    n_rows_total, chunk = uid2.shape
    batch = n_rows_total * chunk
    embed = user_table.shape[1]
    rows_per_w = batch // NUM_WORKERS
    nch = rows_per_w // chunk

    mesh = plsc.VectorSubcoreMesh(core_axis_name="c", subcore_axis_name="s")

    @functools.partial(
        pl.kernel,
        mesh=mesh,
        out_type=(
            jax.ShapeDtypeStruct((batch, embed), jnp.float32),
            jax.ShapeDtypeStruct((batch, embed), jnp.float32),
        ),
        scratch_types=[
            pltpu.VMEM((nch, chunk), jnp.int32),
            pltpu.VMEM((nch, chunk), jnp.int32),
            pltpu.VMEM((rows_per_w, embed), jnp.float32),
            pltpu.SemaphoreType.DMA,
        ],
    )
    def gather_kernel(uid_hbm, iid_hbm, ut_hbm, it_hbm, out_u, out_i,
                      uidx_v, iidx_v, rows_v, sem):
        wid = lax.axis_index("s") * 2 + lax.axis_index("c")
        base = wid * rows_per_w
        idx_row = wid * nch
        pltpu.sync_copy(uid_hbm.at[pl.ds(idx_row, nch)], uidx_v)
        pltpu.sync_copy(iid_hbm.at[pl.ds(idx_row, nch)], iidx_v)
        cps = [
            pltpu.async_copy(ut_hbm.at[uidx_v.at[j]],
                             rows_v.at[pl.ds(j * chunk, chunk)], sem)
            for j in range(nch)
        ]
        for c in cps:
            c.wait()
        pltpu.sync_copy(rows_v, out_u.at[pl.ds(base, rows_per_w)])
        cps = [
            pltpu.async_copy(it_hbm.at[iidx_v.at[j]],
                             rows_v.at[pl.ds(j * chunk, chunk)], sem)
            for j in range(nch)
        ]
        for c in cps:
            c.wait()
        pltpu.sync_copy(rows_v, out_i.at[pl.ds(base, rows_per_w)])

    return gather_kernel(uid2, iid2, user_table, item_table)


def _mlp_body(ue_ref, ie_ref, w1_ref, b1_ref, w2_ref, b2_ref,
              w3_ref, b3_ref, o_ref):
    embed = ue_ref.shape[1]
    h = jnp.dot(ue_ref[...], w1_ref[0:embed, :],
                preferred_element_type=jnp.float32)
    h = h + jnp.dot(ie_ref[...], w1_ref[embed:2 * embed, :],
                    preferred_element_type=jnp.float32)
    h1 = jnp.maximum(h + b1_ref[...], 0.0)
    h2 = jnp.maximum(
        jnp.dot(h1, w2_ref[...], preferred_element_type=jnp.float32)
        + b2_ref[...], 0.0)
    ot = lax.dot_general(w3_ref[...], h2, (((0,), (1,)), ((), ())),
                         preferred_element_type=jnp.float32)
    o_ref[...] = jnp.maximum(ot + b3_ref[...], 0.0)[None]


def _mlp(ue, ie, w1, b1, w2, b2, w3, b3, block=2048):
    batch, embed = ue.shape
    grid = batch // block
    full = lambda shape: pl.BlockSpec(shape, lambda i: (0, 0))
    return pl.pallas_call(
        _mlp_body,
        grid=(grid,),
        in_specs=[
            pl.BlockSpec((block, embed), lambda i: (i, 0)),
            pl.BlockSpec((block, embed), lambda i: (i, 0)),
            full(w1.shape),
            full(b1.shape),
            full(w2.shape),
            full(b2.shape),
            full(w3.shape),
            full(b3.shape),
        ],
        out_specs=pl.BlockSpec((1, 1, block), lambda i: (i, 0, 0)),
        out_shape=jax.ShapeDtypeStruct((grid, 1, block), jnp.float32),
    )(ue, ie, w1, b1, w2, b2, w3, b3)


NUM_CHUNKS = 2


def kernel(user_id, item_id, user_table, item_table, W1, b1, W2, b2, W3, b3):
    batch = user_id.shape[0]
    uid2 = user_id.astype(jnp.int32).reshape(batch // IDX_CHUNK, IDX_CHUNK)
    iid2 = item_id.astype(jnp.int32).reshape(batch // IDX_CHUNK, IDX_CHUNK)
    b1r = b1.reshape(1, -1)
    b2r = b2.reshape(1, -1)
    b3r = b3.reshape(1, 1)
    n_rows = uid2.shape[0]
    rows_per_chunk = n_rows // NUM_CHUNKS
    outs = []
    for c in range(NUM_CHUNKS):
        sl = slice(c * rows_per_chunk, (c + 1) * rows_per_chunk)
        ue, ie = _gather_pair(uid2[sl], iid2[sl], user_table, item_table)
        outs.append(_mlp(ue, ie, W1, b1r, W2, b2r, W3, b3r))
    out2d = jnp.concatenate(outs, axis=0) if NUM_CHUNKS > 1 else outs[0]
    return out2d.reshape(-1)


# speedup vs baseline: 1.1030x; 1.0451x over previous
import functools

import jax
import jax.numpy as jnp
from jax import lax
from jax.experimental import pallas as pl
from jax.experimental.pallas import tpu as pltpu
from jax.experimental.pallas import tpu_sc as plsc

NUM_WORKERS = 32
IDX_CHUNK = 128


def _gather_pair(uid2, iid2, user_table, item_table):
    n_rows_total, chunk = uid2.shape
    batch = n_rows_total * chunk
    embed = user_table.shape[1]
    rows_per_w = batch // NUM_WORKERS
    nch = rows_per_w // chunk

    mesh = plsc.VectorSubcoreMesh(core_axis_name="c", subcore_axis_name="s")

    assert rows_per_w * embed * 4 * 2 <= 500_000, "two staging buffers must fit"

    @functools.partial(
        pl.kernel,
        mesh=mesh,
        out_type=(
            jax.ShapeDtypeStruct((batch, embed), jnp.float32),
            jax.ShapeDtypeStruct((batch, embed), jnp.float32),
        ),
        scratch_types=[
            pltpu.VMEM((nch, chunk), jnp.int32),
            pltpu.VMEM((nch, chunk), jnp.int32),
            pltpu.VMEM((rows_per_w, embed), jnp.float32),
            pltpu.VMEM((rows_per_w, embed), jnp.float32),
            pltpu.SemaphoreType.DMA,
            pltpu.SemaphoreType.DMA,
            pltpu.SemaphoreType.DMA,
        ],
    )
    def gather_kernel(uid_hbm, iid_hbm, ut_hbm, it_hbm, out_u, out_i,
                      uidx_v, iidx_v, rows_u, rows_i, sem_u, sem_i, sem_w):
        wid = lax.axis_index("s") * 2 + lax.axis_index("c")
        base = wid * rows_per_w
        idx_row = wid * nch
        pltpu.sync_copy(uid_hbm.at[pl.ds(idx_row, nch)], uidx_v)
        pltpu.sync_copy(iid_hbm.at[pl.ds(idx_row, nch)], iidx_v)
        cps_u = [
            pltpu.async_copy(ut_hbm.at[uidx_v.at[j]],
                             rows_u.at[pl.ds(j * chunk, chunk)], sem_u)
            for j in range(nch)
        ]
        cps_i = [
            pltpu.async_copy(it_hbm.at[iidx_v.at[j]],
                             rows_i.at[pl.ds(j * chunk, chunk)], sem_i)
            for j in range(nch)
        ]
        for c in cps_u:
            c.wait()
        w_u = pltpu.async_copy(rows_u, out_u.at[pl.ds(base, rows_per_w)],
                               sem_w)
        for c in cps_i:
            c.wait()
        w_i = pltpu.async_copy(rows_i, out_i.at[pl.ds(base, rows_per_w)],
                               sem_w)
        w_u.wait()
        w_i.wait()

    return gather_kernel(uid2, iid2, user_table, item_table)


def _mlp_body(ue_ref, ie_ref, w1_ref, b1_ref, w2_ref, b2_ref,
              w3_ref, b3_ref, o_ref):
    embed = ue_ref.shape[1]
    h = jnp.dot(ue_ref[...], w1_ref[0:embed, :],
                preferred_element_type=jnp.float32)
    h = h + jnp.dot(ie_ref[...], w1_ref[embed:2 * embed, :],
                    preferred_element_type=jnp.float32)
    h1 = jnp.maximum(h + b1_ref[...], 0.0)
    h2 = jnp.maximum(
        jnp.dot(h1, w2_ref[...], preferred_element_type=jnp.float32)
        + b2_ref[...], 0.0)
    ot = lax.dot_general(w3_ref[...], h2, (((0,), (1,)), ((), ())),
                         preferred_element_type=jnp.float32)
    o_ref[...] = jnp.maximum(ot + b3_ref[...], 0.0)[None]


def _mlp(ue, ie, w1, b1, w2, b2, w3, b3, block=2048):
    batch, embed = ue.shape
    grid = batch // block
    full = lambda shape: pl.BlockSpec(shape, lambda i: (0, 0))
    return pl.pallas_call(
        _mlp_body,
        grid=(grid,),
        in_specs=[
            pl.BlockSpec((block, embed), lambda i: (i, 0)),
            pl.BlockSpec((block, embed), lambda i: (i, 0)),
            full(w1.shape),
            full(b1.shape),
            full(w2.shape),
            full(b2.shape),
            full(w3.shape),
            full(b3.shape),
        ],
        out_specs=pl.BlockSpec((1, 1, block), lambda i: (i, 0, 0)),
        out_shape=jax.ShapeDtypeStruct((grid, 1, block), jnp.float32),
    )(ue, ie, w1, b1, w2, b2, w3, b3)


NUM_CHUNKS = 2


def kernel(user_id, item_id, user_table, item_table, W1, b1, W2, b2, W3, b3):
    batch = user_id.shape[0]
    uid2 = user_id.astype(jnp.int32).reshape(batch // IDX_CHUNK, IDX_CHUNK)
    iid2 = item_id.astype(jnp.int32).reshape(batch // IDX_CHUNK, IDX_CHUNK)
    b1r = b1.reshape(1, -1)
    b2r = b2.reshape(1, -1)
    b3r = b3.reshape(1, 1)
    n_rows = uid2.shape[0]
    rows_per_chunk = n_rows // NUM_CHUNKS
    outs = []
    for c in range(NUM_CHUNKS):
        sl = slice(c * rows_per_chunk, (c + 1) * rows_per_chunk)
        ue, ie = _gather_pair(uid2[sl], iid2[sl], user_table, item_table)
        outs.append(_mlp(ue, ie, W1, b1r, W2, b2r, W3, b3r))
    out2d = jnp.concatenate(outs, axis=0) if NUM_CHUNKS > 1 else outs[0]
    return out2d.reshape(-1)


# speedup vs baseline: 1.1095x; 1.0059x over previous
import functools

import jax
import jax.numpy as jnp
from jax import lax
from jax.experimental import pallas as pl
from jax.experimental.pallas import tpu as pltpu
from jax.experimental.pallas import tpu_sc as plsc

NUM_WORKERS = 32
IDX_CHUNK = 128


def _gather_pair(uid2, iid2, user_table, item_table, chunk_idx, num_chunks):
    n_rows_total, chunk = uid2.shape
    batch = (n_rows_total // num_chunks) * chunk
    base_row = chunk_idx * (n_rows_total // num_chunks)
    embed = user_table.shape[1]
    rows_per_w = batch // NUM_WORKERS
    nch = rows_per_w // chunk

    mesh = plsc.VectorSubcoreMesh(core_axis_name="c", subcore_axis_name="s")

    assert rows_per_w * embed * 4 * 2 <= 500_000, "two staging buffers must fit"

    @functools.partial(
        pl.kernel,
        mesh=mesh,
        out_type=(
            jax.ShapeDtypeStruct((batch, embed), jnp.float32),
            jax.ShapeDtypeStruct((batch, embed), jnp.float32),
        ),
        scratch_types=[
            pltpu.VMEM((nch, chunk), jnp.int32),
            pltpu.VMEM((nch, chunk), jnp.int32),
            pltpu.VMEM((rows_per_w, embed), jnp.float32),
            pltpu.VMEM((rows_per_w, embed), jnp.float32),
            pltpu.SemaphoreType.DMA,
            pltpu.SemaphoreType.DMA,
            pltpu.SemaphoreType.DMA,
        ],
    )
    def gather_kernel(uid_hbm, iid_hbm, ut_hbm, it_hbm, out_u, out_i,
                      uidx_v, iidx_v, rows_u, rows_i, sem_u, sem_i, sem_w):
        wid = lax.axis_index("s") * 2 + lax.axis_index("c")
        base = wid * rows_per_w
        idx_row = base_row + wid * nch
        pltpu.sync_copy(uid_hbm.at[pl.ds(idx_row, nch)], uidx_v)
        pltpu.sync_copy(iid_hbm.at[pl.ds(idx_row, nch)], iidx_v)
        cps_u = [
            pltpu.async_copy(ut_hbm.at[uidx_v.at[j]],
                             rows_u.at[pl.ds(j * chunk, chunk)], sem_u)
            for j in range(nch)
        ]
        cps_i = [
            pltpu.async_copy(it_hbm.at[iidx_v.at[j]],
                             rows_i.at[pl.ds(j * chunk, chunk)], sem_i)
            for j in range(nch)
        ]
        for c in cps_u:
            c.wait()
        w_u = pltpu.async_copy(rows_u, out_u.at[pl.ds(base, rows_per_w)],
                               sem_w)
        for c in cps_i:
            c.wait()
        w_i = pltpu.async_copy(rows_i, out_i.at[pl.ds(base, rows_per_w)],
                               sem_w)
        w_u.wait()
        w_i.wait()

    return gather_kernel(uid2, iid2, user_table, item_table)


def _mlp_body(ue_ref, ie_ref, w1_ref, b1_ref, w2_ref, b2_ref,
              w3_ref, b3_ref, o_ref):
    embed = ue_ref.shape[1]
    h = jnp.dot(ue_ref[...], w1_ref[0:embed, :],
                preferred_element_type=jnp.float32)
    h = h + jnp.dot(ie_ref[...], w1_ref[embed:2 * embed, :],
                    preferred_element_type=jnp.float32)
    h1 = jnp.maximum(h + b1_ref[...], 0.0)
    h2 = jnp.maximum(
        jnp.dot(h1, w2_ref[...], preferred_element_type=jnp.float32)
        + b2_ref[...], 0.0)
    ot = lax.dot_general(w3_ref[...], h2, (((0,), (1,)), ((), ())),
                         preferred_element_type=jnp.float32)
    o_ref[...] = jnp.maximum(ot + b3_ref[...], 0.0)[None]


def _mlp(ue, ie, w1, b1, w2, b2, w3, b3, block=2048):
    batch, embed = ue.shape
    grid = batch // block
    full = lambda shape: pl.BlockSpec(shape, lambda i: (0, 0))
    return pl.pallas_call(
        _mlp_body,
        grid=(grid,),
        in_specs=[
            pl.BlockSpec((block, embed), lambda i: (i, 0)),
            pl.BlockSpec((block, embed), lambda i: (i, 0)),
            full(w1.shape),
            full(b1.shape),
            full(w2.shape),
            full(b2.shape),
            full(w3.shape),
            full(b3.shape),
        ],
        out_specs=pl.BlockSpec((1, 1, block), lambda i: (i, 0, 0)),
        out_shape=jax.ShapeDtypeStruct((grid, 1, block), jnp.float32),
    )(ue, ie, w1, b1, w2, b2, w3, b3)


NUM_CHUNKS = 2


def kernel(user_id, item_id, user_table, item_table, W1, b1, W2, b2, W3, b3):
    batch = user_id.shape[0]
    uid2 = user_id.astype(jnp.int32).reshape(batch // IDX_CHUNK, IDX_CHUNK)
    iid2 = item_id.astype(jnp.int32).reshape(batch // IDX_CHUNK, IDX_CHUNK)
    b1r = b1.reshape(1, -1)
    b2r = b2.reshape(1, -1)
    b3r = b3.reshape(1, 1)
    n_rows = uid2.shape[0]
    rows_per_chunk = n_rows // NUM_CHUNKS
    outs = []
    for c in range(NUM_CHUNKS):
        ue, ie = _gather_pair(uid2, iid2, user_table, item_table,
                              c, NUM_CHUNKS)
        outs.append(_mlp(ue, ie, W1, b1r, W2, b2r, W3, b3r))
    out2d = jnp.concatenate(outs, axis=0) if NUM_CHUNKS > 1 else outs[0]
    return out2d.reshape(-1)
